# Initial kernel scaffold; baseline (speedup 1.0000x reference)
import jax
import jax.numpy as jnp
from jax.experimental import pallas as pl


def kernel(x, W0, W1, W2, W3, W4, W5, W6):
    raise NotImplementedError("write your pallas kernel here")


# speedup vs baseline: 2.3429x; 2.3429x over previous
import jax
import jax.numpy as jnp
from jax import lax
from jax.experimental import pallas as pl
from jax.experimental.pallas import tpu as pltpu
from jax.experimental.pallas import tpu_sc as plsc

SEQ = 200
BATCH = 1024
DM = 64
NV = 4
LANES = 16
NC, NS = 2, 16
NW = NC * NS
ROWS_PER_W = BATCH // NW
CH = 72
CHUNKS = ((0, CH), (CH, CH), (2 * CH, SEQ - 2 * CH))
NPAD = 208
TOK = BATCH * SEQ


def _unpack_pair(gb, wrc, t, vbase):
    for c in range(2 * DM // LANES):
        wrc[t, vbase + c // 4, pl.ds((c % 4) * LANES, LANES)] = (
            gb[t, pl.ds(c * LANES, LANES)])


def _body(xT, WL0, WR1, WL2, WR3, WL4, WR5, W6f, out,
          xidx, pidx_s, pidx_f, w6v, wrc, gb01, gb23, gb45, sem):
    wid = lax.axis_index("s") * NC + lax.axis_index("c")

    iota = lax.iota(jnp.int32, LANES)
    for i in range(NPAD // LANES):
        s = iota + (i * LANES)
        pidx_s[pl.ds(i * LANES, LANES)] = jnp.minimum(s, SEQ - 1)
        pidx_f[pl.ds(i * LANES, LANES)] = jnp.clip(s - 149, 0, 50)

    pltpu.sync_copy(W6f, w6v)
    for c, (off, ck) in enumerate(CHUNKS):
        dst = gb45.at[c, pl.ds(0, ck)]
        pltpu.async_copy(WL4.at[pidx_s.at[pl.ds(off, ck)]], dst, sem).wait()
        pltpu.async_copy(WR5.at[pidx_f.at[pl.ds(off, ck)]], dst, sem,
                         add=True).wait()

    def row_step(i, carry):
        r = wid * ROWS_PER_W + i
        t0 = r * SEQ
        for c, (off, ck) in enumerate(CHUNKS):
            for v in range(NV):
                pltpu.sync_copy(xT.at[pl.ds(v * TOK + t0 + off, ck)],
                                xidx.at[pl.ds(v * CH, ck)])
            cps = [
                pltpu.async_copy(WL0.at[xidx.at[pl.ds(0 * CH, ck)]],
                                 gb01.at[pl.ds(0, ck)], sem),
                pltpu.async_copy(WL2.at[xidx.at[pl.ds(2 * CH, ck)]],
                                 gb23.at[pl.ds(0, ck)], sem),
            ]
            for cp in cps:
                cp.wait()
            cps = [
                pltpu.async_copy(WR1.at[xidx.at[pl.ds(1 * CH, ck)]],
                                 gb01.at[pl.ds(0, ck)], sem, add=True),
                pltpu.async_copy(WR3.at[xidx.at[pl.ds(3 * CH, ck)]],
                                 gb23.at[pl.ds(0, ck)], sem, add=True),
            ]
            for cp in cps:
                cp.wait()

            def unpack(t, carry2):
                _unpack_pair(gb01, wrc, t, 0)
                _unpack_pair(gb23, wrc, t, 2)
                _unpack_pair(gb45.at[c], wrc, t, 4)
                hi = off + t >= SEQ - 50
                for cc in range(DM // LANES):
                    lo_vec = w6v[pl.ds(cc * LANES, LANES)]
                    hi_vec = w6v[pl.ds(DM + cc * LANES, LANES)]
                    wrc[t, 6, pl.ds(cc * LANES, LANES)] = jnp.where(
                        hi, hi_vec, lo_vec)
                return carry2
            lax.fori_loop(0, ck, unpack, 0)

            pltpu.async_copy(wrc.at[pl.ds(0, ck)],
                             out.at[r, pl.ds(off, ck)], sem).wait()
        return carry

    lax.fori_loop(0, ROWS_PER_W, row_step, 0)


def _pair(w, side):
    z = jnp.zeros_like(w)
    cols = (w, z) if side == 0 else (z, w)
    return jnp.concatenate(cols, axis=1)


def kernel(x, W0, W1, W2, W3, W4, W5, W6):
    xT = jnp.transpose(x.astype(jnp.int32), (2, 0, 1)).reshape(NV * TOK)
    WL0, WL2, WL4 = _pair(W0, 0), _pair(W2, 0), _pair(W4, 0)
    WR1, WR3, WR5 = _pair(W1, 1), _pair(W3, 1), _pair(W5, 1)
    W6f = W6.reshape(2 * DM)

    mesh = plsc.VectorSubcoreMesh(core_axis_name="c", subcore_axis_name="s")
    f = pl.kernel(
        _body,
        out_type=jax.ShapeDtypeStruct((BATCH, SEQ, 7, DM), jnp.float32),
        mesh=mesh,
        scratch_types=[
            pltpu.VMEM((NV * CH,), jnp.int32),
            pltpu.VMEM((NPAD,), jnp.int32),
            pltpu.VMEM((NPAD,), jnp.int32),
            pltpu.VMEM((2 * DM,), jnp.float32),
            pltpu.VMEM((CH, 7, DM), jnp.float32),
            pltpu.VMEM((CH, 2 * DM), jnp.float32),
            pltpu.VMEM((CH, 2 * DM), jnp.float32),
            pltpu.VMEM((3, CH, 2 * DM), jnp.float32),
            pltpu.SemaphoreType.DMA,
        ],
    )
    return f(xT, WL0, WR1, WL2, WR3, WL4, WR5, W6f)


# speedup vs baseline: 3.6923x; 1.5759x over previous
import jax
import jax.numpy as jnp
from jax import lax
from jax.experimental import pallas as pl
from jax.experimental.pallas import tpu as pltpu
from jax.experimental.pallas import tpu_sc as plsc

SEQ = 200
BATCH = 1024
DM = 64
NV = 4
LANES = 16
NC, NS = 2, 16
NW = NC * NS
ROWS_PER_W = BATCH // NW
CH = 40
NCHUNK = SEQ // CH
WTOK = ROWS_PER_W * SEQ
NPAD = NCHUNK * CH + 0
IPAD = 208
TOK = BATCH * SEQ


def _body(xTc, Wd0, Wd1, Wd2, Wd3, WL4, WR5, W6f, out,
          xidxA, xidxB, pidx_s, pidx_f, w6v, wrc, gbA, gbB, gb45,
          semA, semB, wsem):
    wid = lax.axis_index("s") * NC + lax.axis_index("c")
    tbase = wid * WTOK
    gbase = wid * ROWS_PER_W * NCHUNK
    tables = (Wd0, Wd1, Wd2, Wd3)

    iota = lax.iota(jnp.int32, LANES)
    for i in range(IPAD // LANES):
        s = iota + (i * LANES)
        pidx_s[pl.ds(i * LANES, LANES)] = jnp.minimum(s, SEQ - 1)
        pidx_f[pl.ds(i * LANES, LANES)] = jnp.clip(s - 149, 0, 50)

    pltpu.sync_copy(W6f, w6v)
    w6lo = [w6v[pl.ds(c * LANES, LANES)] for c in range(DM // LANES)]
    w6hi = [w6v[pl.ds(DM + c * LANES, LANES)] for c in range(DM // LANES)]
    for c in range(NCHUNK):
        dst = gb45.at[c]
        pltpu.async_copy(WL4.at[pidx_s.at[pl.ds(c * CH, CH)]],
                         dst, semA).wait()
        pltpu.async_copy(WR5.at[pidx_f.at[pl.ds(c * CH, CH)]],
                         dst, semA, add=True).wait()

    def fire(i, cpos, xi, gb, sem):
        gid = gbase + i * NCHUNK + cpos
        pltpu.sync_copy(xTc.at[pl.ds(gid * (NV * CH), NV * CH)], xi)
        for v in range(NV):
            pltpu.async_copy(tables[v].at[xi.at[pl.ds(v * CH, CH)]],
                             gb.at[v], sem)

    def drain_g(gb, sem):
        for v in range(NV):
            pltpu.make_async_copy(Wd0.at[pl.ds(0, CH)], gb.at[v], sem).wait()

    def drain_w():
        pltpu.make_async_copy(wrc, out.at[pl.ds(0, CH)], wsem).wait()

    def unpack_data(gb):
        def step(t, carry):
            for v in range(NV):
                for c in range(DM // LANES):
                    wrc[t, v, pl.ds(c * LANES, LANES)] = (
                        gb[v, t, pl.ds(c * LANES, LANES)])
            return carry
        lax.fori_loop(0, CH, step, 0)

    def write(i, cpos):
        pltpu.async_copy(wrc, out.at[pl.ds(tbase + i * SEQ + cpos * CH, CH)],
                         wsem)

    pltpu.async_copy(wrc, out.at[pl.ds(tbase, CH)], wsem)

    for cpos in range(NCHUNK):
        drain_w()
        lo_all = (cpos + 1) * CH <= SEQ - 50
        hi_all = cpos * CH >= SEQ - 50

        def pos_step(t, carry):
            for c in range(2 * DM // LANES):
                wrc[t, 4 + c // 4, pl.ds((c % 4) * LANES, LANES)] = (
                    gb45[cpos, t, pl.ds(c * LANES, LANES)])
            for c in range(DM // LANES):
                if lo_all:
                    vec = w6lo[c]
                elif hi_all:
                    vec = w6hi[c]
                else:
                    vec = jnp.where(cpos * CH + t >= SEQ - 50,
                                    w6hi[c], w6lo[c])
                wrc[t, 6, pl.ds(c * LANES, LANES)] = vec
            return carry
        lax.fori_loop(0, CH, pos_step, 0)

        pltpu.async_copy(wrc, out.at[pl.ds(tbase + cpos * CH, CH)], wsem)
        fire(jnp.int32(0), cpos, xidxA, gbA, semA)
        fire(jnp.int32(1), cpos, xidxB, gbB, semB)

        def seg_body(j, carry):
            i = 2 * j
            drain_g(gbA, semA)
            drain_w()
            unpack_data(gbA)
            write(i, cpos)
            fire(jnp.minimum(i + 2, ROWS_PER_W - 1), cpos, xidxA, gbA, semA)
            drain_g(gbB, semB)
            drain_w()
            unpack_data(gbB)
            write(i + 1, cpos)
            fire(jnp.minimum(i + 3, ROWS_PER_W - 1), cpos, xidxB, gbB, semB)
            return carry
        lax.fori_loop(0, ROWS_PER_W // 2, seg_body, 0)

        drain_g(gbA, semA)
        drain_g(gbB, semB)

    drain_w()


def _pair(w, side):
    z = jnp.zeros_like(w)
    cols = (w, z) if side == 0 else (z, w)
    return jnp.concatenate(cols, axis=1)


def kernel(x, W0, W1, W2, W3, W4, W5, W6):
    xTc = jnp.transpose(
        x.astype(jnp.int32).reshape(BATCH, NCHUNK, CH, NV),
        (0, 1, 3, 2)).reshape(-1)
    Wd = [jnp.concatenate([w, w], axis=1) for w in (W0, W1, W2, W3)]
    WL4, WR5 = _pair(W4, 0), _pair(W5, 1)
    W6f = W6.reshape(2 * DM)

    mesh = plsc.VectorSubcoreMesh(core_axis_name="c", subcore_axis_name="s")
    f = pl.kernel(
        _body,
        out_type=jax.ShapeDtypeStruct((TOK, 7, DM), jnp.float32),
        mesh=mesh,
        scratch_types=[
            pltpu.VMEM((NV * CH,), jnp.int32),
            pltpu.VMEM((NV * CH,), jnp.int32),
            pltpu.VMEM((IPAD,), jnp.int32),
            pltpu.VMEM((IPAD,), jnp.int32),
            pltpu.VMEM((2 * DM,), jnp.float32),
            pltpu.VMEM((CH, 7, DM), jnp.float32),
            pltpu.VMEM((NV, CH, 2 * DM), jnp.float32),
            pltpu.VMEM((NV, CH, 2 * DM), jnp.float32),
            pltpu.VMEM((NCHUNK, CH, 2 * DM), jnp.float32),
            pltpu.SemaphoreType.DMA,
            pltpu.SemaphoreType.DMA,
            pltpu.SemaphoreType.DMA,
        ],
    )
    out = f(xTc, *Wd, WL4, WR5, W6f)
    return out.reshape(BATCH, SEQ, 7, DM)
